# baseline (speedup 1.0000x reference)
import numpy as np
import jax
import jax.numpy as jnp
from jax import lax
from jax.experimental import pallas as pl
from jax.experimental.pallas import tpu as pltpu
from jax.experimental.pallas import tpu_sc as plsc

_N = 100000
_D = 128
_B = 4096

_TILE_N = 2048
_NPAD = 100352
_NSTEPS = _NPAD // _TILE_N
_GRP = 7
_R = 8
_NB = _B // _R

_K0 = np.uint32(2247515013)
_K1 = np.uint32(2545468385)
_KS2 = np.uint32(_K0 ^ _K1 ^ np.uint32(0x1BD11BDA))
_KS = (int(_K0), int(_K1), int(_KS2))
_INJ = tuple(
    (np.uint32(_KS[i % 3]), np.uint32((_KS[(i + 1) % 3] + i) & 0xFFFFFFFF))
    for i in range(1, 6))

_TINY = np.float32(np.finfo(np.float32).tiny)
_ROT_A = (13, 15, 26, 6)
_ROT_B = (17, 29, 16, 24)


def _threefry_bits(x1):
    x0 = jnp.full(x1.shape, _K0, dtype=jnp.uint32)
    for i, (c0, c1) in enumerate(_INJ):
        rots = _ROT_A if i % 2 == 0 else _ROT_B
        for r in rots:
            x0 = x0 + x1
            x1 = (x1 << jnp.uint32(r)) | (x1 >> jnp.uint32(32 - r))
            x1 = x1 ^ x0
        x0 = x0 + c0
        x1 = x1 + c1
    return x0 ^ x1


def _gumbel_from_bits(bits):
    fb = (bits >> jnp.uint32(9)) | jnp.uint32(0x3F800000)
    f = lax.bitcast_convert_type(fb, jnp.float32) - jnp.float32(1.0)
    u = jnp.maximum(f, _TINY)
    return -jnp.log(-jnp.log(u))


def _sampler_body(w_ref, idx_ref):
    b0 = pl.program_id(0)
    row = lax.broadcasted_iota(jnp.uint32, (_R, _TILE_N), 0)
    col = lax.broadcasted_iota(jnp.uint32, (_R, _TILE_N), 1)
    px1 = ((jnp.uint32(b0) * jnp.uint32(_R) + row) * jnp.uint32(_N)
           + col + jnp.uint32(_K1))

    def step(t, carry):
        bv, bt = carry
        n0 = t * _TILE_N
        g = _gumbel_from_bits(_threefry_bits(px1 + n0.astype(jnp.uint32)))
        wt = w_ref[0:1, pl.ds(n0, _TILE_N)]
        c = g + wt
        mask = c > bv
        bv = jnp.where(mask, c, bv)
        bt = jnp.where(mask, t, bt)
        return bv, bt

    init = (
        jnp.full((_R, _TILE_N), -jnp.inf, dtype=jnp.float32),
        jnp.zeros((_R, _TILE_N), dtype=jnp.int32),
    )
    bv, bt = lax.fori_loop(0, _NSTEPS, step, init, unroll=7)

    coli = lax.broadcasted_iota(jnp.int32, (_R, _TILE_N), 1)
    ncand = bt * _TILE_N + coli
    m = jnp.max(bv, axis=1, keepdims=True)
    big = jnp.int32(np.int32(2**31 - 1))
    idx = jnp.min(jnp.where(bv == m, ncand, big), axis=1, keepdims=True)
    idx_ref[0, :, :] = idx


def _lse_body(w_ref, out_ref):
    w = w_ref[...]
    m = jnp.max(w)
    s = jnp.sum(jnp.exp(w - m))
    out_ref[...] = jnp.broadcast_to(m + jnp.log(s), (1, 16))


_NC = 2
_NS = 16
_NW = _NC * _NS
_BPW = _B // _NW


def _gather_body(table_ref, w_hbm_ref, lse_ref, idx_ref,
                 out_ref, logw_ref,
                 idx_v, rows_v, wv, logw_v, lse_v, sem, sem2):
    wid = lax.axis_index("s") * _NC + lax.axis_index("c")
    base = wid * _BPW
    pltpu.sync_copy(idx_ref.at[pl.ds(base, _BPW)], idx_v)
    c1 = pltpu.async_copy(table_ref.at[idx_v], rows_v, sem)
    c2 = pltpu.async_copy(w_hbm_ref.at[idx_v], wv, sem2)
    pltpu.sync_copy(lse_ref, lse_v)
    c2.wait()
    lv = lse_v[...]
    for j in range(_BPW // 16):
        logw_v[pl.ds(j * 16, 16)] = wv[pl.ds(j * 16, 16)] - lv
    c1.wait()
    pltpu.sync_copy(rows_v, out_ref.at[pl.ds(base, _BPW)])
    pltpu.sync_copy(logw_v, logw_ref.at[pl.ds(base, _BPW)])


_B_SC = 1248
_B_TC = _B - _B_SC
_NB_TC = _B_TC // _R
_NB_SC = _B_SC // _R
_RPW = _B_SC // _NW


def _scbits_body(out_ref, buf):
    wid = lax.axis_index("s") * _NC + lax.axis_index("c")
    iota16 = lax.iota(jnp.int32, 16).astype(jnp.uint32)

    def row_loop(rr, _):
        r = wid * _RPW + rr
        rowbase = ((_B_TC + r).astype(jnp.uint32) * jnp.uint32(_N)
                   + jnp.uint32(_K1) + iota16)

        def body(v, _):
            pv = rowbase + (v * 16).astype(jnp.uint32)
            buf[pl.ds(v * 16, 16)] = _threefry_bits(pv)
            return 0

        lax.fori_loop(0, _NPAD // 16, body, 0, unroll=8)
        pltpu.sync_copy(buf, out_ref.at[r])
        return 0

    lax.fori_loop(0, _RPW, row_loop, 0)


def _finisher_body(w_ref, bits_ref, idx_ref):
    def step(t, carry):
        bv, bt = carry
        n0 = t * _TILE_N
        g = _gumbel_from_bits(bits_ref[:, pl.ds(n0, _TILE_N)])
        wt = w_ref[0:1, pl.ds(n0, _TILE_N)]
        c = g + wt
        mask = c > bv
        bv = jnp.where(mask, c, bv)
        bt = jnp.where(mask, t, bt)
        return bv, bt

    init = (
        jnp.full((_R, _TILE_N), -jnp.inf, dtype=jnp.float32),
        jnp.zeros((_R, _TILE_N), dtype=jnp.int32),
    )
    bv, bt = lax.fori_loop(0, _NSTEPS, step, init, unroll=7)

    coli = lax.broadcasted_iota(jnp.int32, (_R, _TILE_N), 1)
    ncand = bt * _TILE_N + coli
    m = jnp.max(bv, axis=1, keepdims=True)
    big = jnp.int32(np.int32(2**31 - 1))
    idx = jnp.min(jnp.where(bv == m, ncand, big), axis=1, keepdims=True)
    idx_ref[0, :, :] = idx


def kernel(data, weights):
    w_pad = jnp.pad(
        weights.reshape(1, _N), ((0, 0), (0, _NPAD - _N)),
        constant_values=-np.inf)

    lse = pl.pallas_call(
        _lse_body,
        out_shape=jax.ShapeDtypeStruct((1, 16), jnp.float32),
        in_specs=[pl.BlockSpec((1, _NPAD), lambda: (0, 0))],
        out_specs=pl.BlockSpec((1, 16), lambda: (0, 0)),
    )(w_pad)

    scbits = pl.kernel(
        _scbits_body,
        out_type=jax.ShapeDtypeStruct((_B_SC, _NPAD), jnp.uint32),
        mesh=plsc.VectorSubcoreMesh(core_axis_name="c", subcore_axis_name="s"),
        scratch_types=[pltpu.VMEM((_NPAD,), jnp.uint32)],
    )()

    idx3 = pl.pallas_call(
        _sampler_body,
        grid=(_NB_TC,),
        out_shape=jax.ShapeDtypeStruct((_NB_TC, _R, 1), jnp.int32),
        in_specs=[pl.BlockSpec((1, _NPAD), lambda i: (0, 0))],
        out_specs=pl.BlockSpec((1, _R, 1), lambda i: (i, 0, 0)),
        compiler_params=pltpu.CompilerParams(
            dimension_semantics=("parallel",)),
    )(w_pad)

    idx3b = pl.pallas_call(
        _finisher_body,
        grid=(_NB_SC,),
        out_shape=jax.ShapeDtypeStruct((_NB_SC, _R, 1), jnp.int32),
        in_specs=[
            pl.BlockSpec((1, _NPAD), lambda i: (0, 0)),
            pl.BlockSpec((_R, _NPAD), lambda i: (i, 0)),
        ],
        out_specs=pl.BlockSpec((1, _R, 1), lambda i: (i, 0, 0)),
        compiler_params=pltpu.CompilerParams(
            dimension_semantics=("parallel",)),
    )(w_pad, scbits)

    indices = jnp.concatenate(
        [idx3.reshape(_B_TC), idx3b.reshape(_B_SC)])

    mesh = plsc.VectorSubcoreMesh(core_axis_name="c", subcore_axis_name="s")
    gather = pl.kernel(
        _gather_body,
        out_type=(
            jax.ShapeDtypeStruct((_B, _D), jnp.float32),
            jax.ShapeDtypeStruct((_B,), jnp.float32),
        ),
        mesh=mesh,
        scratch_types=[
            pltpu.VMEM((_BPW,), jnp.int32),
            pltpu.VMEM((_BPW, _D), jnp.float32),
            pltpu.VMEM((_BPW,), jnp.float32),
            pltpu.VMEM((_BPW,), jnp.float32),
            pltpu.VMEM((16,), jnp.float32),
            pltpu.SemaphoreType.DMA,
            pltpu.SemaphoreType.DMA,
        ],
    )
    batch, logw = gather(data, weights, lse.reshape(16), indices)
    return (batch, logw, indices)


# speedup vs baseline: 1.0522x; 1.0522x over previous
import numpy as np
import jax
import jax.numpy as jnp
from jax import lax
from jax.experimental import pallas as pl
from jax.experimental.pallas import tpu as pltpu
from jax.experimental.pallas import tpu_sc as plsc

_N = 100000
_D = 128
_B = 4096

_TILE_N = 2048
_NPAD = 100352
_NSTEPS = _NPAD // _TILE_N
_GRP = 7
_R = 8
_NB = _B // _R

_K0 = np.uint32(2247515013)
_K1 = np.uint32(2545468385)
_KS2 = np.uint32(_K0 ^ _K1 ^ np.uint32(0x1BD11BDA))
_KS = (int(_K0), int(_K1), int(_KS2))
_INJ = tuple(
    (np.uint32(_KS[i % 3]), np.uint32((_KS[(i + 1) % 3] + i) & 0xFFFFFFFF))
    for i in range(1, 6))

_TINY = np.float32(np.finfo(np.float32).tiny)
_ROT_A = (13, 15, 26, 6)
_ROT_B = (17, 29, 16, 24)


def _threefry_bits(x1):
    x0 = jnp.full(x1.shape, _K0, dtype=jnp.uint32)
    for i, (c0, c1) in enumerate(_INJ):
        rots = _ROT_A if i % 2 == 0 else _ROT_B
        for r in rots:
            x0 = x0 + x1
            x1 = (x1 << jnp.uint32(r)) | (x1 >> jnp.uint32(32 - r))
            x1 = x1 ^ x0
        x0 = x0 + c0
        x1 = x1 + c1
    return x0 ^ x1


def _gumbel_from_bits(bits):
    fb = (bits >> jnp.uint32(9)) | jnp.uint32(0x3F800000)
    f = lax.bitcast_convert_type(fb, jnp.float32) - jnp.float32(1.0)
    u = jnp.maximum(f, _TINY)
    return -jnp.log(-jnp.log(u))


def _sampler_body(w_ref, idx_ref):
    b0 = pl.program_id(0)
    row = lax.broadcasted_iota(jnp.uint32, (_R, _TILE_N), 0)
    col = lax.broadcasted_iota(jnp.uint32, (_R, _TILE_N), 1)
    px1 = ((jnp.uint32(b0) * jnp.uint32(_R) + row) * jnp.uint32(_N)
           + col + jnp.uint32(_K1))

    def step(t, carry):
        bv, bt = carry
        n0 = t * _TILE_N
        g = _gumbel_from_bits(_threefry_bits(px1 + n0.astype(jnp.uint32)))
        wt = w_ref[0:1, pl.ds(n0, _TILE_N)]
        c = g + wt
        mask = c > bv
        bv = jnp.where(mask, c, bv)
        bt = jnp.where(mask, t, bt)
        return bv, bt

    init = (
        jnp.full((_R, _TILE_N), -jnp.inf, dtype=jnp.float32),
        jnp.zeros((_R, _TILE_N), dtype=jnp.int32),
    )
    bv, bt = lax.fori_loop(0, _NSTEPS, step, init, unroll=7)

    coli = lax.broadcasted_iota(jnp.int32, (_R, _TILE_N), 1)
    ncand = bt * _TILE_N + coli
    m = jnp.max(bv, axis=1, keepdims=True)
    big = jnp.int32(np.int32(2**31 - 1))
    idx = jnp.min(jnp.where(bv == m, ncand, big), axis=1, keepdims=True)
    idx_ref[0, :, :] = idx


def _lse_body(w_ref, out_ref):
    w = w_ref[...]
    m = jnp.max(w)
    s = jnp.sum(jnp.exp(w - m))
    out_ref[...] = jnp.broadcast_to(m + jnp.log(s), (1, 16))


_NC = 2
_NS = 16
_NW = _NC * _NS
_BPW = _B // _NW


def _gather_body(table_ref, w_hbm_ref, lse_ref, idx_ref,
                 out_ref, logw_ref,
                 idx_v, rows_v, wv, logw_v, lse_v, sem, sem2):
    wid = lax.axis_index("s") * _NC + lax.axis_index("c")
    base = wid * _BPW
    pltpu.sync_copy(idx_ref.at[pl.ds(base, _BPW)], idx_v)
    c1 = pltpu.async_copy(table_ref.at[idx_v], rows_v, sem)
    c2 = pltpu.async_copy(w_hbm_ref.at[idx_v], wv, sem2)
    pltpu.sync_copy(lse_ref, lse_v)
    c2.wait()
    lv = lse_v[...]
    for j in range(_BPW // 16):
        logw_v[pl.ds(j * 16, 16)] = wv[pl.ds(j * 16, 16)] - lv
    c1.wait()
    pltpu.sync_copy(rows_v, out_ref.at[pl.ds(base, _BPW)])
    pltpu.sync_copy(logw_v, logw_ref.at[pl.ds(base, _BPW)])


_B_SC = 1184
_B_TC = _B - _B_SC
_NB_TC = _B_TC // _R
_NB_SC = _B_SC // _R
_RPW = _B_SC // _NW


def _scbits_body(out_ref, buf):
    wid = lax.axis_index("s") * _NC + lax.axis_index("c")
    iota16 = lax.iota(jnp.int32, 16).astype(jnp.uint32)

    def row_loop(rr, _):
        r = wid * _RPW + rr
        rowbase = ((_B_TC + r).astype(jnp.uint32) * jnp.uint32(_N)
                   + jnp.uint32(_K1) + iota16)

        def body(v, _):
            pv = rowbase + (v * 16).astype(jnp.uint32)
            buf[pl.ds(v * 16, 16)] = _threefry_bits(pv)
            return 0

        lax.fori_loop(0, _NPAD // 16, body, 0, unroll=8)
        pltpu.sync_copy(buf, out_ref.at[r])
        return 0

    lax.fori_loop(0, _RPW, row_loop, 0)


def _finisher_body(w_ref, bits_ref, idx_ref):
    def step(t, carry):
        bv, bt = carry
        n0 = t * _TILE_N
        g = _gumbel_from_bits(bits_ref[:, pl.ds(n0, _TILE_N)])
        wt = w_ref[0:1, pl.ds(n0, _TILE_N)]
        c = g + wt
        mask = c > bv
        bv = jnp.where(mask, c, bv)
        bt = jnp.where(mask, t, bt)
        return bv, bt

    init = (
        jnp.full((_R, _TILE_N), -jnp.inf, dtype=jnp.float32),
        jnp.zeros((_R, _TILE_N), dtype=jnp.int32),
    )
    bv, bt = lax.fori_loop(0, _NSTEPS, step, init, unroll=7)

    coli = lax.broadcasted_iota(jnp.int32, (_R, _TILE_N), 1)
    ncand = bt * _TILE_N + coli
    m = jnp.max(bv, axis=1, keepdims=True)
    big = jnp.int32(np.int32(2**31 - 1))
    idx = jnp.min(jnp.where(bv == m, ncand, big), axis=1, keepdims=True)
    idx_ref[0, :, :] = idx


def kernel(data, weights):
    w_pad = jnp.pad(
        weights.reshape(1, _N), ((0, 0), (0, _NPAD - _N)),
        constant_values=-np.inf)

    lse = pl.pallas_call(
        _lse_body,
        out_shape=jax.ShapeDtypeStruct((1, 16), jnp.float32),
        in_specs=[pl.BlockSpec((1, _NPAD), lambda: (0, 0))],
        out_specs=pl.BlockSpec((1, 16), lambda: (0, 0)),
    )(w_pad)

    scbits = pl.kernel(
        _scbits_body,
        out_type=jax.ShapeDtypeStruct((_B_SC, _NPAD), jnp.uint32),
        mesh=plsc.VectorSubcoreMesh(core_axis_name="c", subcore_axis_name="s"),
        scratch_types=[pltpu.VMEM((_NPAD,), jnp.uint32)],
    )()

    idx3 = pl.pallas_call(
        _sampler_body,
        grid=(_NB_TC,),
        out_shape=jax.ShapeDtypeStruct((_NB_TC, _R, 1), jnp.int32),
        in_specs=[pl.BlockSpec((1, _NPAD), lambda i: (0, 0))],
        out_specs=pl.BlockSpec((1, _R, 1), lambda i: (i, 0, 0)),
        compiler_params=pltpu.CompilerParams(
            dimension_semantics=("parallel",)),
    )(w_pad)

    idx3b = pl.pallas_call(
        _finisher_body,
        grid=(_NB_SC,),
        out_shape=jax.ShapeDtypeStruct((_NB_SC, _R, 1), jnp.int32),
        in_specs=[
            pl.BlockSpec((1, _NPAD), lambda i: (0, 0)),
            pl.BlockSpec((_R, _NPAD), lambda i: (i, 0)),
        ],
        out_specs=pl.BlockSpec((1, _R, 1), lambda i: (i, 0, 0)),
        compiler_params=pltpu.CompilerParams(
            dimension_semantics=("parallel",)),
    )(w_pad, scbits)

    indices = jnp.concatenate(
        [idx3.reshape(_B_TC), idx3b.reshape(_B_SC)])

    mesh = plsc.VectorSubcoreMesh(core_axis_name="c", subcore_axis_name="s")
    gather = pl.kernel(
        _gather_body,
        out_type=(
            jax.ShapeDtypeStruct((_B, _D), jnp.float32),
            jax.ShapeDtypeStruct((_B,), jnp.float32),
        ),
        mesh=mesh,
        scratch_types=[
            pltpu.VMEM((_BPW,), jnp.int32),
            pltpu.VMEM((_BPW, _D), jnp.float32),
            pltpu.VMEM((_BPW,), jnp.float32),
            pltpu.VMEM((_BPW,), jnp.float32),
            pltpu.VMEM((16,), jnp.float32),
            pltpu.SemaphoreType.DMA,
            pltpu.SemaphoreType.DMA,
        ],
    )
    batch, logw = gather(data, weights, lse.reshape(16), indices)
    return (batch, logw, indices)
